# baseline (speedup 1.0000x reference)
import functools

import jax
import jax.numpy as jnp
import numpy as np
from jax import lax
from jax.experimental import pallas as pl
from jax.experimental.pallas import tpu as pltpu
from jax.experimental.pallas import tpu_sc as plsc

_N = 10000
_E = 320000
_F = 128
_AVG_LOG = float(np.log(33.0))

_NC = 2
_NS = 16
_NW = _NC * _NS
_EPT = _E // _NW
_CHM = 200


def _mm_kernel(x_ref, w_ref, b_ref, o_ref):
    o_ref[...] = jnp.dot(x_ref[...], w_ref[...],
                         preferred_element_type=jnp.float32) + b_ref[...]


def _mm(x, w, b, block_rows):
    m, k = x.shape
    n = w.shape[1]
    assert m % block_rows == 0, (m, block_rows)
    return pl.pallas_call(
        _mm_kernel,
        grid=(m // block_rows,),
        in_specs=[
            pl.BlockSpec((block_rows, k), lambda i: (i, 0)),
            pl.BlockSpec((k, n), lambda i: (0, 0)),
            pl.BlockSpec((n,), lambda i: (0,)),
        ],
        out_specs=pl.BlockSpec((block_rows, n), lambda i: (i, 0)),
        out_shape=jax.ShapeDtypeStruct((m, n), jnp.float32),
    )(x, w, b)


def _edge_map_body(relu, a_hbm, b_hbm, c_hbm, ai_hbm, bi_hbm, out_hbm,
                   ai_v, bi_v, a_rows, b_rows, c_rows, t_rows, sem1, sem2):
    wid = lax.axis_index("s") * _NC + lax.axis_index("c")
    base = wid * _EPT

    def chunk(k, carry):
        cb = base + k * _CHM
        pltpu.sync_copy(ai_hbm.at[pl.ds(cb, _CHM)], ai_v)
        pltpu.sync_copy(bi_hbm.at[pl.ds(cb, _CHM)], bi_v)
        pltpu.sync_copy(c_hbm.at[pl.ds(cb, _CHM)], c_rows)
        cp1 = pltpu.async_copy(a_hbm.at[ai_v], a_rows, sem1)
        cp2 = pltpu.async_copy(b_hbm.at[bi_v], b_rows, sem2)
        cp1.wait()
        cp2.wait()

        def edge(j, carry2):
            for g in range(_F // 16):
                sl = pl.ds(g * 16, 16)
                t = a_rows[j, sl] + b_rows[j, sl] + c_rows[j, sl]
                if relu:
                    t = jnp.maximum(t, 0.0)
                t_rows[j, sl] = t
            return carry2

        lax.fori_loop(0, _CHM, edge, 0, unroll=2)
        pltpu.sync_copy(t_rows, out_hbm.at[pl.ds(cb, _CHM)])
        return carry

    lax.fori_loop(0, _EPT // _CHM, chunk, 0)


def _edge_map(a, b, c, ai, bi, relu):
    mesh = plsc.VectorSubcoreMesh(core_axis_name="c", subcore_axis_name="s")
    return pl.kernel(
        functools.partial(_edge_map_body, relu),
        mesh=mesh,
        out_type=jax.ShapeDtypeStruct((_E, _F), jnp.float32),
        scratch_types=[
            pltpu.VMEM((_CHM,), jnp.int32),
            pltpu.VMEM((_CHM,), jnp.int32),
            pltpu.VMEM((_CHM, _F), jnp.float32),
            pltpu.VMEM((_CHM, _F), jnp.float32),
            pltpu.VMEM((_CHM, _F), jnp.float32),
            pltpu.VMEM((_CHM, _F), jnp.float32),
            pltpu.SemaphoreType.DMA,
            pltpu.SemaphoreType.DMA,
        ],
    )(a, b, c, ai, bi)


_CHA = 256
_MAXCH = _E // _CHA
_BIG = 3.0e38


def _agg_body(a_hbm, b_hbm, c_hbm, src_hbm, dst_hbm, estart_hbm,
              agg_hbm, estart_v, dst_v, src_v, a_buf, b_buf, c_buf,
              out_buf, sem):
    wid = lax.axis_index("s") * _NC + lax.axis_index("c")
    pltpu.sync_copy(estart_hbm, estart_v)
    ev = estart_v[pl.ds(wid, 16)]
    e0 = ev[0]
    e1 = ev[1]
    NG = _F // 16

    def load_chunk(cb):
        cb = pl.multiple_of(cb, 8)
        pltpu.sync_copy(dst_hbm.at[pl.ds(cb, _CHA)], dst_v.at[pl.ds(0, _CHA)])
        pltpu.sync_copy(src_hbm.at[pl.ds(cb, _CHA)], src_v)
        pltpu.async_copy(b_hbm.at[src_v], b_buf, sem).wait()
        pltpu.sync_copy(c_hbm.at[pl.ds(cb, _CHA)], c_buf)

    W = 4 * _F

    def init_window():
        def iw(r, c):
            z = jnp.zeros((16,), jnp.float32)
            b = jnp.full((16,), _BIG, jnp.float32)
            for g in range(NG):
                out_buf[pl.ds(r * W + g * 16, 16)] = z
                out_buf[pl.ds(r * W + _F + g * 16, 16)] = z
                out_buf[pl.ds(r * W + 2 * _F + g * 16, 16)] = b
                out_buf[pl.ds(r * W + 3 * _F + g * 16, 16)] = -b
            return c
        lax.fori_loop(0, 16, iw, 0)

    def load_awin(wb):
        pltpu.sync_copy(a_hbm.at[pl.ds(pl.multiple_of(wb, 8), 16)], a_buf)

    cb0 = jnp.minimum(e0 & jnp.int32(~7), jnp.int32(_E - _CHA))
    load_chunk(cb0)
    node0 = dst_v[pl.ds(e0 - cb0, 16)][0]
    wb0 = node0 & jnp.int32(~15)
    load_awin(wb0)
    init_window()

    zero = jnp.zeros((16,), jnp.float32)
    acc0 = ([zero] * NG, [zero] * NG,
            [jnp.full((16,), _BIG, jnp.float32)] * NG,
            [jnp.full((16,), -_BIG, jnp.float32)] * NG)

    def chunk(k, carry):
        cb = jnp.minimum(cb0 + k * _CHA, jnp.int32(_E - _CHA))
        active = (cb0 + k * _CHA) < e1

        @pl.when(active & (k > 0))
        def _():
            load_chunk(cb)

        def edge(j, carry2):
            lo, node, wb, s, q, mn, mx = carry2
            eidx = cb + j
            valid = (eidx >= lo) & (eidx < e1)
            nj = jnp.where(valid, dst_v[pl.ds(j, 16)][0], node)
            changed = nj != node
            nwb = nj & jnp.int32(~15)
            adv = changed & (nwb != wb)

            @pl.when(changed)
            def _():
                r = node - wb
                for g in range(NG):
                    out_buf[pl.ds(r * W + g * 16, 16)] = s[g]
                    out_buf[pl.ds(r * W + _F + g * 16, 16)] = q[g]
                    out_buf[pl.ds(r * W + 2 * _F + g * 16, 16)] = mn[g]
                    out_buf[pl.ds(r * W + 3 * _F + g * 16, 16)] = mx[g]

            @pl.when(adv)
            def _():
                pltpu.sync_copy(
                    out_buf,
                    agg_hbm.at[pl.ds(pl.multiple_of(wb * W, 8), 16 * W)])
                init_window()
                load_awin(nwb)

            wb2 = jnp.where(adv, nwb, wb)
            node2 = jnp.where(changed, nj, node)
            r2 = node2 - wb2
            s2, q2, mn2, mx2 = [], [], [], []
            for g in range(NG):
                sl = pl.ds(g * 16, 16)
                m = a_buf[r2, sl] + b_buf[j, sl] + c_buf[j, sl]
                ms = jnp.where(valid, m, 0.0)
                mv = jnp.where(valid, m, _BIG)
                mw = jnp.where(valid, m, -_BIG)
                s2.append(jnp.where(changed, ms, s[g] + ms))
                q2.append(jnp.where(changed, ms * m, q[g] + ms * m))
                mn2.append(jnp.where(changed, mv, jnp.minimum(mn[g], mv)))
                mx2.append(jnp.where(changed, mw, jnp.maximum(mx[g], mw)))
            return (lo, node2, wb2, s2, q2, mn2, mx2)

        lo, node, wb, s, q, mn, mx = carry
        carry2 = lax.fori_loop(0, _CHA, edge,
                               (lo, node, wb, s, q, mn, mx))
        lo, node, wb, s, q, mn, mx = carry2
        lo = jnp.where(active, jnp.maximum(lo, cb + _CHA), lo)
        return (lo, node, wb, s, q, mn, mx)

    lo, node, wb, s, q, mn, mx = lax.fori_loop(
        0, _MAXCH, chunk, (e0, node0, wb0) + acc0)

    @pl.when(e1 > e0)
    def _():
        r = node - wb
        for g in range(NG):
            out_buf[pl.ds(r * W + g * 16, 16)] = s[g]
            out_buf[pl.ds(r * W + _F + g * 16, 16)] = q[g]
            out_buf[pl.ds(r * W + 2 * _F + g * 16, 16)] = mn[g]
            out_buf[pl.ds(r * W + 3 * _F + g * 16, 16)] = mx[g]
        pltpu.sync_copy(out_buf,
                        agg_hbm.at[pl.ds(pl.multiple_of(wb * W, 8), 16 * W)])


def _segment_agg(a, b, c, src_s, dst_s, estart_p):
    mesh = plsc.VectorSubcoreMesh(core_axis_name="c", subcore_axis_name="s")
    return pl.kernel(
        _agg_body,
        mesh=mesh,
        out_type=jax.ShapeDtypeStruct((_N * 4 * _F,), jnp.float32),
        scratch_types=[
            pltpu.VMEM((56,), jnp.int32),
            pltpu.VMEM((_CHA + 16,), jnp.int32),
            pltpu.VMEM((_CHA,), jnp.int32),
            pltpu.VMEM((16, _F), jnp.float32),
            pltpu.VMEM((_CHA, _F), jnp.float32),
            pltpu.VMEM((_CHA, _F), jnp.float32),
            pltpu.VMEM((16 * 4 * _F,), jnp.float32),
            pltpu.SemaphoreType.DMA,
        ],
    )(a, b, c, src_s, dst_s, estart_p)


def kernel(x, edge_index, edge_attr, pos_edge_index, pos_edge_attr,
           neg_edge_index, neg_edge_attr, node_W, node_b, edge_W, edge_b,
           preW, preb, postW, postb, linW, linb, bn_g, bn_b, e1W, e1b,
           e2W, e2b):
    F = _F
    src, dst = edge_index[0], edge_index[1]
    n = x.shape[0]

    perm = jnp.argsort(dst)
    src, dst = src[perm], dst[perm]
    edge_attr = edge_attr[perm]
    offsets = jnp.searchsorted(dst, jnp.arange(n + 1, dtype=jnp.int32),
                               method='scan_unrolled').astype(jnp.int32)
    cnt = (offsets[1:] - offsets[:-1]).astype(jnp.float32)
    tgt = (jnp.arange(_NW + 1, dtype=jnp.int32) * (_E // _NW)).astype(jnp.int32)
    nstart = jnp.searchsorted(offsets, tgt, method='scan_unrolled')
    nstart = jnp.clip(((nstart + 8) // 16) * 16, 0, n).astype(jnp.int32)
    nstart = nstart.at[0].set(0).at[_NW].set(n)
    estart = offsets[nstart]
    estart_p = jnp.concatenate([estart, jnp.zeros((23,), jnp.int32)])

    x = _mm(x, node_W, node_b, 1000)
    ea = _mm(edge_attr, edge_W, edge_b, 2000)
    pea = _mm(pos_edge_attr, edge_W, edge_b, 1000)
    nea = _mm(neg_edge_attr, edge_W, edge_b, 1000)

    denom = jnp.clip(cnt, 1.0)[:, None]
    has = (cnt > 0)[:, None]
    amp = jnp.log(denom + 1.0) / _AVG_LOG
    att = _AVG_LOG / jnp.log(denom + 1.0)
    zerob = jnp.zeros((F,), jnp.float32)

    for i in range(2):
        A = _mm(x, preW[i][:F], zerob, 1000)
        B = _mm(x, preW[i][F:2 * F], zerob, 1000)
        C = _mm(ea, preW[i][2 * F:], preb[i], 2000)
        agg4 = _segment_agg(A, B, C, src, dst, estart_p).reshape(n, 4 * F)
        s1 = agg4[:, :F]
        s2 = agg4[:, F:2 * F]
        mean = jnp.where(has, s1 / denom, 0.0)
        var = jnp.where(has, s2 / denom - mean ** 2, 0.0)
        std = jnp.sqrt(jnp.maximum(var, 0.0) + 1e-5)
        mn = jnp.where(has, agg4[:, 2 * F:3 * F], 0.0)
        mx = jnp.where(has, agg4[:, 3 * F:], 0.0)
        agg = jnp.concatenate([mean, mn, mx, std], axis=-1)
        P0 = _mm(agg, postW[i][F:F + 4 * F], zerob, 1000)
        P1 = _mm(agg, postW[i][F + 4 * F:F + 8 * F], zerob, 1000)
        P2 = _mm(agg, postW[i][F + 8 * F:], zerob, 1000)
        out = _mm(x, postW[i][:F], postb[i], 1000) + P0 + amp * P1 + att * P2
        c = _mm(out, linW[i], linb[i], 1000)
        mu = c.mean(0)
        v = ((c - mu) ** 2).mean(0)
        cbn = (c - mu) / jnp.sqrt(v + 1e-5) * bn_g[i] + bn_b[i]
        x = (x + jax.nn.relu(cbn)) / 2.0
        S = _mm(x, e1W[i][:F], zerob, 1000)
        D = _mm(x, e1W[i][F:2 * F], zerob, 1000)
        G = _mm(ea, e1W[i][2 * F:], e1b[i], 2000)
        T = _edge_map(S, D, G, src, dst, relu=True)
        ea = ea + _mm(T, e2W[i], e2b[i], 2000) * 0.5

    return (x, pea, nea)


# speedup vs baseline: 6.9330x; 6.9330x over previous
import functools

import jax
import jax.numpy as jnp
import numpy as np
from jax import lax
from jax.experimental import pallas as pl
from jax.experimental.pallas import tpu as pltpu
from jax.experimental.pallas import tpu_sc as plsc

_N = 10000
_E = 320000
_F = 128
_AVG_LOG = float(np.log(33.0))

_NC = 2
_NS = 16
_NW = _NC * _NS
_EPT = _E // _NW
_CHM = 200


def _mm_kernel(x_ref, w_ref, b_ref, o_ref):
    o_ref[...] = jnp.dot(x_ref[...], w_ref[...],
                         preferred_element_type=jnp.float32) + b_ref[...]


def _mm(x, w, b, block_rows):
    m, k = x.shape
    n = w.shape[1]
    assert m % block_rows == 0, (m, block_rows)
    return pl.pallas_call(
        _mm_kernel,
        grid=(m // block_rows,),
        in_specs=[
            pl.BlockSpec((block_rows, k), lambda i: (i, 0)),
            pl.BlockSpec((k, n), lambda i: (0, 0)),
            pl.BlockSpec((n,), lambda i: (0,)),
        ],
        out_specs=pl.BlockSpec((block_rows, n), lambda i: (i, 0)),
        out_shape=jax.ShapeDtypeStruct((m, n), jnp.float32),
    )(x, w, b)


def _edge_map_body(relu, a_hbm, b_hbm, c_hbm, ai_hbm, bi_hbm, out_hbm,
                   ai_v, bi_v, a_rows, b_rows, c_rows, t_rows, sem1, sem2):
    wid = lax.axis_index("s") * _NC + lax.axis_index("c")
    base = wid * _EPT

    def chunk(k, carry):
        cb = base + k * _CHM
        pltpu.sync_copy(ai_hbm.at[pl.ds(cb, _CHM)], ai_v)
        pltpu.sync_copy(bi_hbm.at[pl.ds(cb, _CHM)], bi_v)
        pltpu.sync_copy(c_hbm.at[pl.ds(cb, _CHM)], c_rows)
        cp1 = pltpu.async_copy(a_hbm.at[ai_v], a_rows, sem1)
        cp2 = pltpu.async_copy(b_hbm.at[bi_v], b_rows, sem2)
        cp1.wait()
        cp2.wait()

        def edge(j, carry2):
            for g in range(_F // 16):
                sl = pl.ds(g * 16, 16)
                t = a_rows[j, sl] + b_rows[j, sl] + c_rows[j, sl]
                if relu:
                    t = jnp.maximum(t, 0.0)
                t_rows[j, sl] = t
            return carry2

        lax.fori_loop(0, _CHM, edge, 0, unroll=2)
        pltpu.sync_copy(t_rows, out_hbm.at[pl.ds(cb, _CHM)])
        return carry

    lax.fori_loop(0, _EPT // _CHM, chunk, 0)


def _edge_map(a, b, c, ai, bi, relu):
    mesh = plsc.VectorSubcoreMesh(core_axis_name="c", subcore_axis_name="s")
    return pl.kernel(
        functools.partial(_edge_map_body, relu),
        mesh=mesh,
        out_type=jax.ShapeDtypeStruct((_E, _F), jnp.float32),
        scratch_types=[
            pltpu.VMEM((_CHM,), jnp.int32),
            pltpu.VMEM((_CHM,), jnp.int32),
            pltpu.VMEM((_CHM, _F), jnp.float32),
            pltpu.VMEM((_CHM, _F), jnp.float32),
            pltpu.VMEM((_CHM, _F), jnp.float32),
            pltpu.VMEM((_CHM, _F), jnp.float32),
            pltpu.SemaphoreType.DMA,
            pltpu.SemaphoreType.DMA,
        ],
    )(a, b, c, ai, bi)


_CHA = 256
_MAXCH = _E // _CHA
_BIG = 3.0e38


def _agg_body(a_hbm, b_hbm, c_hbm, src_hbm, dst_hbm, estart_hbm,
              agg_hbm, estart_v, dst_v, src_v, a_buf, b_buf, c_buf,
              out_buf, sem):
    wid = lax.axis_index("s") * _NC + lax.axis_index("c")
    pltpu.sync_copy(estart_hbm, estart_v)
    ev = estart_v[pl.ds(wid, 16)]
    e0 = ev[0]
    e1 = ev[1]
    NG = _F // 16
    W = 4 * _F

    def load_chunk(cb):
        cb = pl.multiple_of(cb, 8)
        pltpu.sync_copy(dst_hbm.at[pl.ds(cb, _CHA)], dst_v.at[pl.ds(0, _CHA)])
        pltpu.sync_copy(src_hbm.at[pl.ds(cb, _CHA)], src_v)
        pltpu.async_copy(b_hbm.at[src_v], b_buf, sem).wait()
        pltpu.sync_copy(c_hbm.at[pl.ds(cb, _CHA)], c_buf)

    def init_window():
        def iw(r, c):
            z = jnp.zeros((16,), jnp.float32)
            b = jnp.full((16,), _BIG, jnp.float32)
            for g in range(NG):
                out_buf[pl.ds(r * W + g * 16, 16)] = z
                out_buf[pl.ds(r * W + _F + g * 16, 16)] = z
                out_buf[pl.ds(r * W + 2 * _F + g * 16, 16)] = b
                out_buf[pl.ds(r * W + 3 * _F + g * 16, 16)] = -b
            return c
        lax.fori_loop(0, 16, iw, 0)

    def flush_window(wb):
        pltpu.sync_copy(out_buf,
                        agg_hbm.at[pl.ds(pl.multiple_of(wb * W, 8), 16 * W)])

    def load_awin(wb):
        pltpu.sync_copy(a_hbm.at[pl.ds(pl.multiple_of(wb, 8), 16)], a_buf)

    cb0 = jnp.minimum(e0 & jnp.int32(~7), jnp.int32(_E - _CHA))
    load_chunk(cb0)
    wb0 = dst_v[pl.ds(e0 - cb0, 16)][0] & jnp.int32(~15)
    load_awin(wb0)
    init_window()

    def chunk(k, carry):
        cb = jnp.minimum(cb0 + k * _CHA, jnp.int32(_E - _CHA))
        active = (cb0 + k * _CHA) < e1

        def run(c):
            lo, wb = c

            @pl.when(k > 0)
            def _():
                load_chunk(cb)

            def edge(j, wb):
                eidx = cb + j
                valid = (eidx >= lo) & (eidx < e1)
                nj = jnp.where(valid, dst_v[pl.ds(j, 16)][0], wb)
                nwb = nj & jnp.int32(~15)
                adv = nwb != wb

                @pl.when(adv)
                def _():
                    flush_window(wb)
                    init_window()
                    load_awin(nwb)

                wb2 = jnp.where(adv, nwb, wb)
                r = nj - wb2
                for g in range(NG):
                    sl = pl.ds(g * 16, 16)
                    m = a_buf[r, sl] + b_buf[j, sl] + c_buf[j, sl]
                    ms = jnp.where(valid, m, 0.0)
                    mv = jnp.where(valid, m, _BIG)
                    mw = jnp.where(valid, m, -_BIG)
                    o0 = pl.ds(r * W + g * 16, 16)
                    o1 = pl.ds(r * W + _F + g * 16, 16)
                    o2 = pl.ds(r * W + 2 * _F + g * 16, 16)
                    o3 = pl.ds(r * W + 3 * _F + g * 16, 16)
                    out_buf[o0] = out_buf[o0] + ms
                    out_buf[o1] = out_buf[o1] + ms * m
                    out_buf[o2] = jnp.minimum(out_buf[o2], mv)
                    out_buf[o3] = jnp.maximum(out_buf[o3], mw)
                return wb2

            wb = lax.fori_loop(0, _CHA, edge, wb)
            return (jnp.maximum(lo, cb + _CHA), wb)

        return lax.cond(active, run, lambda c: c, carry)

    lo, wb = lax.fori_loop(0, _MAXCH, chunk, (e0, wb0))

    @pl.when(e1 > e0)
    def _():
        flush_window(wb)


def _segment_agg(a, b, c, src_s, dst_s, estart_p):
    mesh = plsc.VectorSubcoreMesh(core_axis_name="c", subcore_axis_name="s")
    return pl.kernel(
        _agg_body,
        mesh=mesh,
        out_type=jax.ShapeDtypeStruct((_N * 4 * _F,), jnp.float32),
        scratch_types=[
            pltpu.VMEM((56,), jnp.int32),
            pltpu.VMEM((_CHA + 16,), jnp.int32),
            pltpu.VMEM((_CHA,), jnp.int32),
            pltpu.VMEM((16, _F), jnp.float32),
            pltpu.VMEM((_CHA, _F), jnp.float32),
            pltpu.VMEM((_CHA, _F), jnp.float32),
            pltpu.VMEM((16 * 4 * _F,), jnp.float32),
            pltpu.SemaphoreType.DMA,
        ],
    )(a, b, c, src_s, dst_s, estart_p)


def kernel(x, edge_index, edge_attr, pos_edge_index, pos_edge_attr,
           neg_edge_index, neg_edge_attr, node_W, node_b, edge_W, edge_b,
           preW, preb, postW, postb, linW, linb, bn_g, bn_b, e1W, e1b,
           e2W, e2b):
    F = _F
    src, dst = edge_index[0], edge_index[1]
    n = x.shape[0]

    perm = jnp.argsort(dst)
    src, dst = src[perm], dst[perm]
    edge_attr = edge_attr[perm]
    offsets = jnp.searchsorted(dst, jnp.arange(n + 1, dtype=jnp.int32),
                               method='scan_unrolled').astype(jnp.int32)
    cnt = (offsets[1:] - offsets[:-1]).astype(jnp.float32)
    tgt = (jnp.arange(_NW + 1, dtype=jnp.int32) * (_E // _NW)).astype(jnp.int32)
    nstart = jnp.searchsorted(offsets, tgt, method='scan_unrolled')
    nstart = jnp.clip(((nstart + 8) // 16) * 16, 0, n).astype(jnp.int32)
    nstart = nstart.at[0].set(0).at[_NW].set(n)
    estart = offsets[nstart]
    estart_p = jnp.concatenate([estart, jnp.zeros((23,), jnp.int32)])

    x = _mm(x, node_W, node_b, 1000)
    ea = _mm(edge_attr, edge_W, edge_b, 2000)
    pea = _mm(pos_edge_attr, edge_W, edge_b, 1000)
    nea = _mm(neg_edge_attr, edge_W, edge_b, 1000)

    denom = jnp.clip(cnt, 1.0)[:, None]
    has = (cnt > 0)[:, None]
    amp = jnp.log(denom + 1.0) / _AVG_LOG
    att = _AVG_LOG / jnp.log(denom + 1.0)
    zerob = jnp.zeros((F,), jnp.float32)

    for i in range(2):
        A = _mm(x, preW[i][:F], zerob, 1000)
        B = _mm(x, preW[i][F:2 * F], zerob, 1000)
        C = _mm(ea, preW[i][2 * F:], preb[i], 2000)
        agg4 = _segment_agg(A, B, C, src, dst, estart_p).reshape(n, 4 * F)
        s1 = agg4[:, :F]
        s2 = agg4[:, F:2 * F]
        mean = jnp.where(has, s1 / denom, 0.0)
        var = jnp.where(has, s2 / denom - mean ** 2, 0.0)
        std = jnp.sqrt(jnp.maximum(var, 0.0) + 1e-5)
        mn = jnp.where(has, agg4[:, 2 * F:3 * F], 0.0)
        mx = jnp.where(has, agg4[:, 3 * F:], 0.0)
        agg = jnp.concatenate([mean, mn, mx, std], axis=-1)
        P0 = _mm(agg, postW[i][F:F + 4 * F], zerob, 1000)
        P1 = _mm(agg, postW[i][F + 4 * F:F + 8 * F], zerob, 1000)
        P2 = _mm(agg, postW[i][F + 8 * F:], zerob, 1000)
        out = _mm(x, postW[i][:F], postb[i], 1000) + P0 + amp * P1 + att * P2
        c = _mm(out, linW[i], linb[i], 1000)
        mu = c.mean(0)
        v = ((c - mu) ** 2).mean(0)
        cbn = (c - mu) / jnp.sqrt(v + 1e-5) * bn_g[i] + bn_b[i]
        x = (x + jax.nn.relu(cbn)) / 2.0
        S = _mm(x, e1W[i][:F], zerob, 1000)
        D = _mm(x, e1W[i][F:2 * F], zerob, 1000)
        G = _mm(ea, e1W[i][2 * F:], e1b[i], 2000)
        T = _edge_map(S, D, G, src, dst, relu=True)
        ea = ea + _mm(T, e2W[i], e2b[i], 2000) * 0.5

    return (x, pea, nea)


# speedup vs baseline: 7.1258x; 1.0278x over previous
import functools

import jax
import jax.numpy as jnp
import numpy as np
from jax import lax
from jax.experimental import pallas as pl
from jax.experimental.pallas import tpu as pltpu
from jax.experimental.pallas import tpu_sc as plsc

_N = 10000
_E = 320000
_F = 128
_AVG_LOG = float(np.log(33.0))

_NC = 2
_NS = 16
_NW = _NC * _NS
_EPT = _E // _NW
_CHM = 200


def _mm_kernel(x_ref, w_ref, b_ref, o_ref):
    o_ref[...] = jnp.dot(x_ref[...], w_ref[...],
                         preferred_element_type=jnp.float32) + b_ref[...]


def _mm(x, w, b, block_rows):
    m, k = x.shape
    n = w.shape[1]
    assert m % block_rows == 0, (m, block_rows)
    return pl.pallas_call(
        _mm_kernel,
        grid=(m // block_rows,),
        in_specs=[
            pl.BlockSpec((block_rows, k), lambda i: (i, 0)),
            pl.BlockSpec((k, n), lambda i: (0, 0)),
            pl.BlockSpec((n,), lambda i: (0,)),
        ],
        out_specs=pl.BlockSpec((block_rows, n), lambda i: (i, 0)),
        out_shape=jax.ShapeDtypeStruct((m, n), jnp.float32),
    )(x, w, b)


def _edge_map_body(relu, a_hbm, b_hbm, c_hbm, ai_hbm, bi_hbm, out_hbm,
                   ai_v, bi_v, a_rows, b_rows, c_rows, t_rows, sem1, sem2,
                   sem3):
    wid = lax.axis_index("s") * _NC + lax.axis_index("c")
    base = wid * _EPT

    def chunk(k, carry):
        cb = base + k * _CHM
        i1 = pltpu.async_copy(ai_hbm.at[pl.ds(cb, _CHM)], ai_v, sem1)
        i2 = pltpu.async_copy(bi_hbm.at[pl.ds(cb, _CHM)], bi_v, sem2)
        i3 = pltpu.async_copy(c_hbm.at[pl.ds(cb, _CHM)], c_rows, sem3)
        i1.wait()
        cp1 = pltpu.async_copy(a_hbm.at[ai_v], a_rows, sem1)
        i2.wait()
        cp2 = pltpu.async_copy(b_hbm.at[bi_v], b_rows, sem2)
        i3.wait()
        cp1.wait()
        cp2.wait()

        def edge(j, carry2):
            for g in range(_F // 16):
                sl = pl.ds(g * 16, 16)
                t = a_rows[j, sl] + b_rows[j, sl] + c_rows[j, sl]
                if relu:
                    t = jnp.maximum(t, 0.0)
                t_rows[j, sl] = t
            return carry2

        lax.fori_loop(0, _CHM, edge, 0, unroll=2)
        pltpu.sync_copy(t_rows, out_hbm.at[pl.ds(cb, _CHM)])
        return carry

    lax.fori_loop(0, _EPT // _CHM, chunk, 0)


def _edge_map(a, b, c, ai, bi, relu):
    mesh = plsc.VectorSubcoreMesh(core_axis_name="c", subcore_axis_name="s")
    return pl.kernel(
        functools.partial(_edge_map_body, relu),
        mesh=mesh,
        out_type=jax.ShapeDtypeStruct((_E, _F), jnp.float32),
        scratch_types=[
            pltpu.VMEM((_CHM,), jnp.int32),
            pltpu.VMEM((_CHM,), jnp.int32),
            pltpu.VMEM((_CHM, _F), jnp.float32),
            pltpu.VMEM((_CHM, _F), jnp.float32),
            pltpu.VMEM((_CHM, _F), jnp.float32),
            pltpu.VMEM((_CHM, _F), jnp.float32),
            pltpu.SemaphoreType.DMA,
            pltpu.SemaphoreType.DMA,
            pltpu.SemaphoreType.DMA,
        ],
    )(a, b, c, ai, bi)


_CHA = 400
_MAXCH = _E // _CHA
_BIG = 3.0e38


def _agg_body(a_hbm, b_hbm, c_hbm, src_hbm, dst_hbm, estart_hbm,
              agg_hbm, estart_v, dst_v, src_v, a_buf, b_buf, c_buf,
              out_buf, sem, sem2, sem3):
    wid = lax.axis_index("s") * _NC + lax.axis_index("c")
    pltpu.sync_copy(estart_hbm, estart_v)
    ev = estart_v[pl.ds(wid, 16)]
    e0 = ev[0]
    e1 = ev[1]
    NG = _F // 16
    W = 4 * _F

    def load_chunk(cb):
        cb = pl.multiple_of(cb, 8)
        i1 = pltpu.async_copy(dst_hbm.at[pl.ds(cb, _CHA)],
                              dst_v.at[pl.ds(0, _CHA)], sem)
        i2 = pltpu.async_copy(src_hbm.at[pl.ds(cb, _CHA)], src_v, sem2)
        i3 = pltpu.async_copy(c_hbm.at[pl.ds(cb, _CHA)], c_buf, sem3)
        i2.wait()
        g = pltpu.async_copy(b_hbm.at[src_v], b_buf, sem2)
        i1.wait()
        i3.wait()
        g.wait()

    def init_window():
        def iw(r, c):
            z = jnp.zeros((16,), jnp.float32)
            b = jnp.full((16,), _BIG, jnp.float32)
            for g in range(NG):
                out_buf[pl.ds(r * W + g * 16, 16)] = z
                out_buf[pl.ds(r * W + _F + g * 16, 16)] = z
                out_buf[pl.ds(r * W + 2 * _F + g * 16, 16)] = b
                out_buf[pl.ds(r * W + 3 * _F + g * 16, 16)] = -b
            return c
        lax.fori_loop(0, 16, iw, 0)

    def flush_window(wb):
        pltpu.sync_copy(out_buf,
                        agg_hbm.at[pl.ds(pl.multiple_of(wb * W, 8), 16 * W)])

    def load_awin(wb):
        pltpu.sync_copy(a_hbm.at[pl.ds(pl.multiple_of(wb, 8), 16)], a_buf)

    cb0 = jnp.minimum(e0 & jnp.int32(~7), jnp.int32(_E - _CHA))
    load_chunk(cb0)
    wb0 = dst_v[pl.ds(e0 - cb0, 16)][0] & jnp.int32(~15)
    load_awin(wb0)
    init_window()

    def chunk(k, carry):
        cb = jnp.minimum(cb0 + k * _CHA, jnp.int32(_E - _CHA))
        active = (cb0 + k * _CHA) < e1

        def run(c):
            lo, wb = c

            @pl.when(k > 0)
            def _():
                load_chunk(cb)

            def edge(j, wb):
                eidx = cb + j
                valid = (eidx >= lo) & (eidx < e1)
                nj = jnp.where(valid, dst_v[pl.ds(j, 16)][0], wb)
                nwb = nj & jnp.int32(~15)
                adv = nwb != wb

                @pl.when(adv)
                def _():
                    flush_window(wb)
                    init_window()
                    load_awin(nwb)

                wb2 = jnp.where(adv, nwb, wb)
                r = nj - wb2
                for g in range(NG):
                    sl = pl.ds(g * 16, 16)
                    m = a_buf[r, sl] + b_buf[j, sl] + c_buf[j, sl]
                    ms = jnp.where(valid, m, 0.0)
                    mv = jnp.where(valid, m, _BIG)
                    mw = jnp.where(valid, m, -_BIG)
                    o0 = pl.ds(r * W + g * 16, 16)
                    o1 = pl.ds(r * W + _F + g * 16, 16)
                    o2 = pl.ds(r * W + 2 * _F + g * 16, 16)
                    o3 = pl.ds(r * W + 3 * _F + g * 16, 16)
                    out_buf[o0] = out_buf[o0] + ms
                    out_buf[o1] = out_buf[o1] + ms * m
                    out_buf[o2] = jnp.minimum(out_buf[o2], mv)
                    out_buf[o3] = jnp.maximum(out_buf[o3], mw)
                return wb2

            wb = lax.fori_loop(0, _CHA, edge, wb)
            return (jnp.maximum(lo, cb + _CHA), wb)

        return lax.cond(active, run, lambda c: c, carry)

    lo, wb = lax.fori_loop(0, _MAXCH, chunk, (e0, wb0))

    @pl.when(e1 > e0)
    def _():
        flush_window(wb)


def _segment_agg(a, b, c, src_s, dst_s, estart_p):
    mesh = plsc.VectorSubcoreMesh(core_axis_name="c", subcore_axis_name="s")
    return pl.kernel(
        _agg_body,
        mesh=mesh,
        out_type=jax.ShapeDtypeStruct((_N * 4 * _F,), jnp.float32),
        scratch_types=[
            pltpu.VMEM((56,), jnp.int32),
            pltpu.VMEM((_CHA + 16,), jnp.int32),
            pltpu.VMEM((_CHA,), jnp.int32),
            pltpu.VMEM((16, _F), jnp.float32),
            pltpu.VMEM((_CHA, _F), jnp.float32),
            pltpu.VMEM((_CHA, _F), jnp.float32),
            pltpu.VMEM((16 * 4 * _F,), jnp.float32),
            pltpu.SemaphoreType.DMA,
            pltpu.SemaphoreType.DMA,
            pltpu.SemaphoreType.DMA,
        ],
    )(a, b, c, src_s, dst_s, estart_p)


def kernel(x, edge_index, edge_attr, pos_edge_index, pos_edge_attr,
           neg_edge_index, neg_edge_attr, node_W, node_b, edge_W, edge_b,
           preW, preb, postW, postb, linW, linb, bn_g, bn_b, e1W, e1b,
           e2W, e2b):
    F = _F
    src, dst = edge_index[0], edge_index[1]
    n = x.shape[0]

    perm = jnp.argsort(dst)
    src, dst = src[perm], dst[perm]
    edge_attr = edge_attr[perm]
    offsets = jnp.searchsorted(dst, jnp.arange(n + 1, dtype=jnp.int32),
                               method='scan_unrolled').astype(jnp.int32)
    cnt = (offsets[1:] - offsets[:-1]).astype(jnp.float32)
    tgt = (jnp.arange(_NW + 1, dtype=jnp.int32) * (_E // _NW)).astype(jnp.int32)
    nstart = jnp.searchsorted(offsets, tgt, method='scan_unrolled')
    nstart = jnp.clip(((nstart + 8) // 16) * 16, 0, n).astype(jnp.int32)
    nstart = nstart.at[0].set(0).at[_NW].set(n)
    estart = offsets[nstart]
    estart_p = jnp.concatenate([estart, jnp.zeros((23,), jnp.int32)])

    x = _mm(x, node_W, node_b, 1000)
    ea = _mm(edge_attr, edge_W, edge_b, 2000)
    pea = _mm(pos_edge_attr, edge_W, edge_b, 1000)
    nea = _mm(neg_edge_attr, edge_W, edge_b, 1000)

    denom = jnp.clip(cnt, 1.0)[:, None]
    has = (cnt > 0)[:, None]
    amp = jnp.log(denom + 1.0) / _AVG_LOG
    att = _AVG_LOG / jnp.log(denom + 1.0)
    zerob = jnp.zeros((F,), jnp.float32)

    for i in range(2):
        A = _mm(x, preW[i][:F], zerob, 1000)
        B = _mm(x, preW[i][F:2 * F], zerob, 1000)
        C = _mm(ea, preW[i][2 * F:], preb[i], 2000)
        agg4 = _segment_agg(A, B, C, src, dst, estart_p).reshape(n, 4 * F)
        s1 = agg4[:, :F]
        s2 = agg4[:, F:2 * F]
        mean = jnp.where(has, s1 / denom, 0.0)
        var = jnp.where(has, s2 / denom - mean ** 2, 0.0)
        std = jnp.sqrt(jnp.maximum(var, 0.0) + 1e-5)
        mn = jnp.where(has, agg4[:, 2 * F:3 * F], 0.0)
        mx = jnp.where(has, agg4[:, 3 * F:], 0.0)
        agg = jnp.concatenate([mean, mn, mx, std], axis=-1)
        P0 = _mm(agg, postW[i][F:F + 4 * F], zerob, 1000)
        P1 = _mm(agg, postW[i][F + 4 * F:F + 8 * F], zerob, 1000)
        P2 = _mm(agg, postW[i][F + 8 * F:], zerob, 1000)
        out = _mm(x, postW[i][:F], postb[i], 1000) + P0 + amp * P1 + att * P2
        c = _mm(out, linW[i], linb[i], 1000)
        mu = c.mean(0)
        v = ((c - mu) ** 2).mean(0)
        cbn = (c - mu) / jnp.sqrt(v + 1e-5) * bn_g[i] + bn_b[i]
        x = (x + jax.nn.relu(cbn)) / 2.0
        S = _mm(x, e1W[i][:F], zerob, 1000)
        D = _mm(x, e1W[i][F:2 * F], zerob, 1000)
        G = _mm(ea, e1W[i][2 * F:], e1b[i], 2000)
        T = _edge_map(S, D, G, src, dst, relu=True)
        ea = ea + _mm(T, e2W[i], e2b[i], 2000) * 0.5

    return (x, pea, nea)


# speedup vs baseline: 7.2250x; 1.0139x over previous
import functools

import jax
import jax.numpy as jnp
import numpy as np
from jax import lax
from jax.experimental import pallas as pl
from jax.experimental.pallas import tpu as pltpu
from jax.experimental.pallas import tpu_sc as plsc

_N = 10000
_E = 320000
_F = 128
_AVG_LOG = float(np.log(33.0))

_NC = 2
_NS = 16
_NW = _NC * _NS
_EPT = _E // _NW
_CHM = 200


def _mm_kernel(x_ref, w_ref, b_ref, o_ref):
    o_ref[...] = jnp.dot(x_ref[...], w_ref[...],
                         preferred_element_type=jnp.float32) + b_ref[...]


def _mm(x, w, b, block_rows):
    m, k = x.shape
    n = w.shape[1]
    assert m % block_rows == 0, (m, block_rows)
    return pl.pallas_call(
        _mm_kernel,
        grid=(m // block_rows,),
        in_specs=[
            pl.BlockSpec((block_rows, k), lambda i: (i, 0)),
            pl.BlockSpec((k, n), lambda i: (0, 0)),
            pl.BlockSpec((n,), lambda i: (0,)),
        ],
        out_specs=pl.BlockSpec((block_rows, n), lambda i: (i, 0)),
        out_shape=jax.ShapeDtypeStruct((m, n), jnp.float32),
    )(x, w, b)


def _edge_map_body(relu, a_hbm, b_hbm, c_hbm, ai_hbm, bi_hbm, out_hbm,
                   ai_v, bi_v, a_rows, b_rows, c_rows, t_rows, sem1, sem2,
                   sem3):
    wid = lax.axis_index("s") * _NC + lax.axis_index("c")
    base = wid * _EPT

    def chunk(k, carry):
        cb = base + k * _CHM
        i1 = pltpu.async_copy(ai_hbm.at[pl.ds(cb, _CHM)], ai_v, sem1)
        i2 = pltpu.async_copy(bi_hbm.at[pl.ds(cb, _CHM)], bi_v, sem2)
        i3 = pltpu.async_copy(c_hbm.at[pl.ds(cb, _CHM)], c_rows, sem3)
        i1.wait()
        cp1 = pltpu.async_copy(a_hbm.at[ai_v], a_rows, sem1)
        i2.wait()
        cp2 = pltpu.async_copy(b_hbm.at[bi_v], b_rows, sem2)
        i3.wait()
        cp1.wait()
        cp2.wait()

        def edge(j, carry2):
            for g in range(_F // 16):
                sl = pl.ds(g * 16, 16)
                t = a_rows[j, sl] + b_rows[j, sl] + c_rows[j, sl]
                if relu:
                    t = jnp.maximum(t, 0.0)
                t_rows[j, sl] = t
            return carry2

        lax.fori_loop(0, _CHM, edge, 0, unroll=2)
        pltpu.sync_copy(t_rows, out_hbm.at[pl.ds(cb, _CHM)])
        return carry

    lax.fori_loop(0, _EPT // _CHM, chunk, 0)


def _edge_map(a, b, c, ai, bi, relu):
    mesh = plsc.VectorSubcoreMesh(core_axis_name="c", subcore_axis_name="s")
    return pl.kernel(
        functools.partial(_edge_map_body, relu),
        mesh=mesh,
        out_type=jax.ShapeDtypeStruct((_E, _F), jnp.float32),
        scratch_types=[
            pltpu.VMEM((_CHM,), jnp.int32),
            pltpu.VMEM((_CHM,), jnp.int32),
            pltpu.VMEM((_CHM, _F), jnp.float32),
            pltpu.VMEM((_CHM, _F), jnp.float32),
            pltpu.VMEM((_CHM, _F), jnp.float32),
            pltpu.VMEM((_CHM, _F), jnp.float32),
            pltpu.SemaphoreType.DMA,
            pltpu.SemaphoreType.DMA,
            pltpu.SemaphoreType.DMA,
        ],
    )(a, b, c, ai, bi)


_CHA = 400
_MAXCH = _E // _CHA
_BIG = 3.0e38


def _agg_body(a_hbm, b_hbm, c_hbm, src_hbm, dst_hbm, estart_hbm,
              agg_hbm, estart_v, dst_v, src_v, a_buf, b_buf, c_buf,
              out_buf, sem, sem2, sem3):
    wid = lax.axis_index("s") * _NC + lax.axis_index("c")
    pltpu.sync_copy(estart_hbm, estart_v)
    ev = estart_v[pl.ds(wid, 16)]
    e0 = ev[0]
    e1 = ev[1]
    NG = _F // 16
    W = 4 * _F

    def load_chunk(cb):
        cb = pl.multiple_of(cb, 8)
        i1 = pltpu.async_copy(dst_hbm.at[pl.ds(cb, _CHA)],
                              dst_v.at[pl.ds(0, _CHA)], sem)
        i2 = pltpu.async_copy(src_hbm.at[pl.ds(cb, _CHA)], src_v, sem2)
        i3 = pltpu.async_copy(c_hbm.at[pl.ds(cb, _CHA)], c_buf, sem3)
        i2.wait()
        g = pltpu.async_copy(b_hbm.at[src_v], b_buf, sem2)
        i1.wait()
        i3.wait()
        g.wait()

    def init_window():
        def iw(r, c):
            z = jnp.zeros((16,), jnp.float32)
            b = jnp.full((16,), _BIG, jnp.float32)
            for g in range(NG):
                out_buf[pl.ds(r * W + g * 16, 16)] = z
                out_buf[pl.ds(r * W + _F + g * 16, 16)] = z
                out_buf[pl.ds(r * W + 2 * _F + g * 16, 16)] = b
                out_buf[pl.ds(r * W + 3 * _F + g * 16, 16)] = -b
            return c
        lax.fori_loop(0, 16, iw, 0)

    def flush_window(wb):
        pltpu.sync_copy(out_buf,
                        agg_hbm.at[pl.ds(pl.multiple_of(wb * W, 8), 16 * W)])

    def load_awin(wb):
        pltpu.sync_copy(a_hbm.at[pl.ds(pl.multiple_of(wb, 8), 16)], a_buf)

    cb0 = jnp.minimum(e0 & jnp.int32(~7), jnp.int32(_E - _CHA))
    load_chunk(cb0)
    wb0 = dst_v[pl.ds(e0 - cb0, 16)][0] & jnp.int32(~15)
    load_awin(wb0)
    init_window()

    def chunk(k, carry):
        cb = jnp.minimum(cb0 + k * _CHA, jnp.int32(_E - _CHA))
        active = (cb0 + k * _CHA) < e1

        def run(c):
            lo, wb = c

            @pl.when(k > 0)
            def _():
                load_chunk(cb)

            def edge(j, wb):
                eidx = cb + j
                valid = (eidx >= lo) & (eidx < e1)
                nj = jnp.where(valid, dst_v[pl.ds(j, 16)][0], wb)
                nwb = nj & jnp.int32(~15)
                adv = nwb != wb

                @pl.when(adv)
                def _():
                    flush_window(wb)
                    init_window()
                    load_awin(nwb)

                wb2 = jnp.where(adv, nwb, wb)
                r = nj - wb2
                for g in range(NG):
                    sl = pl.ds(g * 16, 16)
                    m = a_buf[r, sl] + b_buf[j, sl] + c_buf[j, sl]
                    ms = jnp.where(valid, m, 0.0)
                    mv = jnp.where(valid, m, _BIG)
                    mw = jnp.where(valid, m, -_BIG)
                    o0 = pl.ds(r * W + g * 16, 16)
                    o1 = pl.ds(r * W + _F + g * 16, 16)
                    o2 = pl.ds(r * W + 2 * _F + g * 16, 16)
                    o3 = pl.ds(r * W + 3 * _F + g * 16, 16)
                    plsc.addupdate(out_buf.at[o0], ms)
                    plsc.addupdate(out_buf.at[o1], ms * m)
                    out_buf[o2] = jnp.minimum(out_buf[o2], mv)
                    out_buf[o3] = jnp.maximum(out_buf[o3], mw)
                return wb2

            wb = lax.fori_loop(0, _CHA, edge, wb)
            return (jnp.maximum(lo, cb + _CHA), wb)

        return lax.cond(active, run, lambda c: c, carry)

    lo, wb = lax.fori_loop(0, _MAXCH, chunk, (e0, wb0))

    @pl.when(e1 > e0)
    def _():
        flush_window(wb)


def _segment_agg(a, b, c, src_s, dst_s, estart_p):
    mesh = plsc.VectorSubcoreMesh(core_axis_name="c", subcore_axis_name="s")
    return pl.kernel(
        _agg_body,
        mesh=mesh,
        out_type=jax.ShapeDtypeStruct((_N * 4 * _F,), jnp.float32),
        scratch_types=[
            pltpu.VMEM((56,), jnp.int32),
            pltpu.VMEM((_CHA + 16,), jnp.int32),
            pltpu.VMEM((_CHA,), jnp.int32),
            pltpu.VMEM((16, _F), jnp.float32),
            pltpu.VMEM((_CHA, _F), jnp.float32),
            pltpu.VMEM((_CHA, _F), jnp.float32),
            pltpu.VMEM((16 * 4 * _F,), jnp.float32),
            pltpu.SemaphoreType.DMA,
            pltpu.SemaphoreType.DMA,
            pltpu.SemaphoreType.DMA,
        ],
    )(a, b, c, src_s, dst_s, estart_p)


def kernel(x, edge_index, edge_attr, pos_edge_index, pos_edge_attr,
           neg_edge_index, neg_edge_attr, node_W, node_b, edge_W, edge_b,
           preW, preb, postW, postb, linW, linb, bn_g, bn_b, e1W, e1b,
           e2W, e2b):
    F = _F
    src, dst = edge_index[0], edge_index[1]
    n = x.shape[0]

    perm = jnp.argsort(dst)
    src, dst = src[perm], dst[perm]
    edge_attr = edge_attr[perm]
    offsets = jnp.searchsorted(dst, jnp.arange(n + 1, dtype=jnp.int32),
                               method='scan_unrolled').astype(jnp.int32)
    cnt = (offsets[1:] - offsets[:-1]).astype(jnp.float32)
    tgt = (jnp.arange(_NW + 1, dtype=jnp.int32) * (_E // _NW)).astype(jnp.int32)
    nstart = jnp.searchsorted(offsets, tgt, method='scan_unrolled')
    nstart = jnp.clip(((nstart + 8) // 16) * 16, 0, n).astype(jnp.int32)
    nstart = nstart.at[0].set(0).at[_NW].set(n)
    estart = offsets[nstart]
    estart_p = jnp.concatenate([estart, jnp.zeros((23,), jnp.int32)])

    x = _mm(x, node_W, node_b, 1000)
    ea = _mm(edge_attr, edge_W, edge_b, 2000)
    pea = _mm(pos_edge_attr, edge_W, edge_b, 1000)
    nea = _mm(neg_edge_attr, edge_W, edge_b, 1000)

    denom = jnp.clip(cnt, 1.0)[:, None]
    has = (cnt > 0)[:, None]
    amp = jnp.log(denom + 1.0) / _AVG_LOG
    att = _AVG_LOG / jnp.log(denom + 1.0)
    zerob = jnp.zeros((F,), jnp.float32)

    for i in range(2):
        A = _mm(x, preW[i][:F], zerob, 1000)
        B = _mm(x, preW[i][F:2 * F], zerob, 1000)
        C = _mm(ea, preW[i][2 * F:], preb[i], 2000)
        agg4 = _segment_agg(A, B, C, src, dst, estart_p).reshape(n, 4 * F)
        s1 = agg4[:, :F]
        s2 = agg4[:, F:2 * F]
        mean = jnp.where(has, s1 / denom, 0.0)
        var = jnp.where(has, s2 / denom - mean ** 2, 0.0)
        std = jnp.sqrt(jnp.maximum(var, 0.0) + 1e-5)
        mn = jnp.where(has, agg4[:, 2 * F:3 * F], 0.0)
        mx = jnp.where(has, agg4[:, 3 * F:], 0.0)
        agg = jnp.concatenate([mean, mn, mx, std], axis=-1)
        P0 = _mm(agg, postW[i][F:F + 4 * F], zerob, 1000)
        P1 = _mm(agg, postW[i][F + 4 * F:F + 8 * F], zerob, 1000)
        P2 = _mm(agg, postW[i][F + 8 * F:], zerob, 1000)
        out = _mm(x, postW[i][:F], postb[i], 1000) + P0 + amp * P1 + att * P2
        c = _mm(out, linW[i], linb[i], 1000)
        mu = c.mean(0)
        v = ((c - mu) ** 2).mean(0)
        cbn = (c - mu) / jnp.sqrt(v + 1e-5) * bn_g[i] + bn_b[i]
        x = (x + jax.nn.relu(cbn)) / 2.0
        S = _mm(x, e1W[i][:F], zerob, 1000)
        D = _mm(x, e1W[i][F:2 * F], zerob, 1000)
        G = _mm(ea, e1W[i][2 * F:], e1b[i], 2000)
        T = _edge_map(S, D, G, src, dst, relu=True)
        ea = ea + _mm(T, e2W[i], e2b[i], 2000) * 0.5

    return (x, pea, nea)


# speedup vs baseline: 7.4238x; 1.0275x over previous
import functools

import jax
import jax.numpy as jnp
import numpy as np
from jax import lax
from jax.experimental import pallas as pl
from jax.experimental.pallas import tpu as pltpu
from jax.experimental.pallas import tpu_sc as plsc

_N = 10000
_E = 320000
_F = 128
_AVG_LOG = float(np.log(33.0))

_NC = 2
_NS = 16
_NW = _NC * _NS
_EPT = _E // _NW
_CHM = 40


def _mm_kernel(x_ref, w_ref, b_ref, o_ref):
    o_ref[...] = jnp.dot(x_ref[...], w_ref[...],
                         preferred_element_type=jnp.float32) + b_ref[...]


def _mm(x, w, b, block_rows):
    m, k = x.shape
    n = w.shape[1]
    assert m % block_rows == 0, (m, block_rows)
    return pl.pallas_call(
        _mm_kernel,
        grid=(m // block_rows,),
        in_specs=[
            pl.BlockSpec((block_rows, k), lambda i: (i, 0)),
            pl.BlockSpec((k, n), lambda i: (0, 0)),
            pl.BlockSpec((n,), lambda i: (0,)),
        ],
        out_specs=pl.BlockSpec((block_rows, n), lambda i: (i, 0)),
        out_shape=jax.ShapeDtypeStruct((m, n), jnp.float32),
    )(x, w, b)


def _edge_map_body(relu, a_hbm, b_hbm, c_hbm, ai_hbm, bi_hbm, out_hbm,
                   ai_v0, bi_v0, a_r0, b_r0, c_r0, t_r0,
                   ai_v1, bi_v1, a_r1, b_r1, c_r1, t_r1,
                   si0, si1, sg0, sg1, sc0, sc1, so0, so1):
    wid = lax.axis_index("s") * _NC + lax.axis_index("c")
    base = wid * _EPT
    NCH = _EPT // _CHM
    sets = ((ai_v0, bi_v0, a_r0, b_r0, c_r0, t_r0, si0, sg0, sc0, so0),
            (ai_v1, bi_v1, a_r1, b_r1, c_r1, t_r1, si1, sg1, sc1, so1))

    def issue(k, st):
        ai_v, bi_v, a_r, b_r, c_r, t_r, si, sg, sc, so = st
        cb = base + k * _CHM
        i1 = pltpu.async_copy(ai_hbm.at[pl.ds(cb, _CHM)], ai_v, si)
        i2 = pltpu.async_copy(bi_hbm.at[pl.ds(cb, _CHM)], bi_v, sg)
        pltpu.async_copy(c_hbm.at[pl.ds(cb, _CHM)], c_r, sc)
        i1.wait()
        i2.wait()
        pltpu.async_copy(a_hbm.at[ai_v], a_r, si)
        pltpu.async_copy(b_hbm.at[bi_v], b_r, sg)

    def compute(k, st, first):
        ai_v, bi_v, a_r, b_r, c_r, t_r, si, sg, sc, so = st
        cb = base + k * _CHM
        pltpu.make_async_copy(a_hbm.at[ai_v], a_r, si).wait()
        pltpu.make_async_copy(b_hbm.at[bi_v], b_r, sg).wait()
        pltpu.make_async_copy(c_hbm.at[pl.ds(cb, _CHM)], c_r, sc).wait()

        @pl.when(jnp.logical_not(first))
        def _():
            pltpu.make_async_copy(
                t_r, out_hbm.at[pl.ds(cb, _CHM)], so).wait()

        def edge(j, carry2):
            for g in range(_F // 16):
                sl = pl.ds(g * 16, 16)
                t = a_r[j, sl] + b_r[j, sl] + c_r[j, sl]
                if relu:
                    t = jnp.maximum(t, 0.0)
                t_r[j, sl] = t
            return carry2

        lax.fori_loop(0, _CHM, edge, 0, unroll=2)
        pltpu.async_copy(t_r, out_hbm.at[pl.ds(cb, _CHM)], so)

    issue(0, sets[0])

    def pair(p, carry):
        for b in range(2):
            k = p * 2 + b
            nxt = k + 1

            @pl.when(nxt < NCH)
            def _():
                issue(nxt, sets[1 - b])
            compute(k, sets[b], first=(k < 2))
        return carry

    lax.fori_loop(0, NCH // 2, pair, 0)
    pltpu.make_async_copy(
        t_r0, out_hbm.at[pl.ds(base, _CHM)], so0).wait()
    pltpu.make_async_copy(
        t_r1, out_hbm.at[pl.ds(base, _CHM)], so1).wait()


def _edge_map(a, b, c, ai, bi, relu):
    mesh = plsc.VectorSubcoreMesh(core_axis_name="c", subcore_axis_name="s")
    buf = lambda: pltpu.VMEM((_CHM, _F), jnp.float32)
    idx = lambda: pltpu.VMEM((_CHM,), jnp.int32)
    return pl.kernel(
        functools.partial(_edge_map_body, relu),
        mesh=mesh,
        out_type=jax.ShapeDtypeStruct((_E, _F), jnp.float32),
        scratch_types=[
            idx(), idx(), buf(), buf(), buf(), buf(),
            idx(), idx(), buf(), buf(), buf(), buf(),
        ] + [pltpu.SemaphoreType.DMA] * 8,
    )(a, b, c, ai, bi)


_CHA = 400
_MAXCH = _E // _CHA
_BIG = 3.0e38


def _agg_body(a_hbm, b_hbm, c_hbm, src_hbm, dst_hbm, estart_hbm,
              agg_hbm, estart_v, dst_v, src_v, a_buf, b_buf, c_buf,
              out_buf, sem, sem2, sem3):
    wid = lax.axis_index("s") * _NC + lax.axis_index("c")
    pltpu.sync_copy(estart_hbm, estart_v)
    ev = estart_v[pl.ds(wid, 16)]
    e0 = ev[0]
    e1 = ev[1]
    NG = _F // 16
    W = 4 * _F

    def load_chunk(cb):
        cb = pl.multiple_of(cb, 8)
        i1 = pltpu.async_copy(dst_hbm.at[pl.ds(cb, _CHA)],
                              dst_v.at[pl.ds(0, _CHA)], sem)
        i2 = pltpu.async_copy(src_hbm.at[pl.ds(cb, _CHA)], src_v, sem2)
        i3 = pltpu.async_copy(c_hbm.at[pl.ds(cb, _CHA)], c_buf, sem3)
        i2.wait()
        g = pltpu.async_copy(b_hbm.at[src_v], b_buf, sem2)
        i1.wait()
        i3.wait()
        g.wait()

    def init_window():
        def iw(r, c):
            z = jnp.zeros((16,), jnp.float32)
            b = jnp.full((16,), _BIG, jnp.float32)
            for g in range(NG):
                out_buf[pl.ds(r * W + g * 16, 16)] = z
                out_buf[pl.ds(r * W + _F + g * 16, 16)] = z
                out_buf[pl.ds(r * W + 2 * _F + g * 16, 16)] = b
                out_buf[pl.ds(r * W + 3 * _F + g * 16, 16)] = -b
            return c
        lax.fori_loop(0, 16, iw, 0)

    def flush_window(wb):
        pltpu.sync_copy(out_buf,
                        agg_hbm.at[pl.ds(pl.multiple_of(wb * W, 8), 16 * W)])

    def load_awin(wb):
        pltpu.sync_copy(a_hbm.at[pl.ds(pl.multiple_of(wb, 8), 16)], a_buf)

    cb0 = jnp.minimum(e0 & jnp.int32(~7), jnp.int32(_E - _CHA))
    load_chunk(cb0)
    wb0 = dst_v[pl.ds(e0 - cb0, 16)][0] & jnp.int32(~15)
    load_awin(wb0)
    init_window()

    def chunk(k, carry):
        cb = jnp.minimum(cb0 + k * _CHA, jnp.int32(_E - _CHA))
        active = (cb0 + k * _CHA) < e1

        def run(c):
            lo, wb = c

            @pl.when(k > 0)
            def _():
                load_chunk(cb)

            def edge(j, wb):
                eidx = cb + j
                valid = (eidx >= lo) & (eidx < e1)
                nj = jnp.where(valid, dst_v[pl.ds(j, 16)][0], wb)
                nwb = nj & jnp.int32(~15)
                adv = nwb != wb

                @pl.when(adv)
                def _():
                    flush_window(wb)
                    init_window()
                    load_awin(nwb)

                wb2 = jnp.where(adv, nwb, wb)
                r = nj - wb2
                for g in range(NG):
                    sl = pl.ds(g * 16, 16)
                    m = a_buf[r, sl] + b_buf[j, sl] + c_buf[j, sl]
                    ms = jnp.where(valid, m, 0.0)
                    mv = jnp.where(valid, m, _BIG)
                    mw = jnp.where(valid, m, -_BIG)
                    o0 = pl.ds(r * W + g * 16, 16)
                    o1 = pl.ds(r * W + _F + g * 16, 16)
                    o2 = pl.ds(r * W + 2 * _F + g * 16, 16)
                    o3 = pl.ds(r * W + 3 * _F + g * 16, 16)
                    plsc.addupdate(out_buf.at[o0], ms)
                    plsc.addupdate(out_buf.at[o1], ms * m)
                    out_buf[o2] = jnp.minimum(out_buf[o2], mv)
                    out_buf[o3] = jnp.maximum(out_buf[o3], mw)
                return wb2

            wb = lax.fori_loop(0, _CHA, edge, wb)
            return (jnp.maximum(lo, cb + _CHA), wb)

        return lax.cond(active, run, lambda c: c, carry)

    lo, wb = lax.fori_loop(0, _MAXCH, chunk, (e0, wb0))

    @pl.when(e1 > e0)
    def _():
        flush_window(wb)


def _segment_agg(a, b, c, src_s, dst_s, estart_p):
    mesh = plsc.VectorSubcoreMesh(core_axis_name="c", subcore_axis_name="s")
    return pl.kernel(
        _agg_body,
        mesh=mesh,
        out_type=jax.ShapeDtypeStruct((_N * 4 * _F,), jnp.float32),
        scratch_types=[
            pltpu.VMEM((56,), jnp.int32),
            pltpu.VMEM((_CHA + 16,), jnp.int32),
            pltpu.VMEM((_CHA,), jnp.int32),
            pltpu.VMEM((16, _F), jnp.float32),
            pltpu.VMEM((_CHA, _F), jnp.float32),
            pltpu.VMEM((_CHA, _F), jnp.float32),
            pltpu.VMEM((16 * 4 * _F,), jnp.float32),
            pltpu.SemaphoreType.DMA,
            pltpu.SemaphoreType.DMA,
            pltpu.SemaphoreType.DMA,
        ],
    )(a, b, c, src_s, dst_s, estart_p)


def kernel(x, edge_index, edge_attr, pos_edge_index, pos_edge_attr,
           neg_edge_index, neg_edge_attr, node_W, node_b, edge_W, edge_b,
           preW, preb, postW, postb, linW, linb, bn_g, bn_b, e1W, e1b,
           e2W, e2b):
    F = _F
    src, dst = edge_index[0], edge_index[1]
    n = x.shape[0]

    perm = jnp.argsort(dst)
    src, dst = src[perm], dst[perm]
    edge_attr = edge_attr[perm]
    offsets = jnp.searchsorted(dst, jnp.arange(n + 1, dtype=jnp.int32),
                               method='scan_unrolled').astype(jnp.int32)
    cnt = (offsets[1:] - offsets[:-1]).astype(jnp.float32)
    tgt = (jnp.arange(_NW + 1, dtype=jnp.int32) * (_E // _NW)).astype(jnp.int32)
    nstart = jnp.searchsorted(offsets, tgt, method='scan_unrolled')
    nstart = jnp.clip(((nstart + 8) // 16) * 16, 0, n).astype(jnp.int32)
    nstart = nstart.at[0].set(0).at[_NW].set(n)
    estart = offsets[nstart]
    estart_p = jnp.concatenate([estart, jnp.zeros((23,), jnp.int32)])

    x = _mm(x, node_W, node_b, 1000)
    ea = _mm(edge_attr, edge_W, edge_b, 2000)
    pea = _mm(pos_edge_attr, edge_W, edge_b, 1000)
    nea = _mm(neg_edge_attr, edge_W, edge_b, 1000)

    denom = jnp.clip(cnt, 1.0)[:, None]
    has = (cnt > 0)[:, None]
    amp = jnp.log(denom + 1.0) / _AVG_LOG
    att = _AVG_LOG / jnp.log(denom + 1.0)
    zerob = jnp.zeros((F,), jnp.float32)

    for i in range(2):
        A = _mm(x, preW[i][:F], zerob, 1000)
        B = _mm(x, preW[i][F:2 * F], zerob, 1000)
        C = _mm(ea, preW[i][2 * F:], preb[i], 2000)
        agg4 = _segment_agg(A, B, C, src, dst, estart_p).reshape(n, 4 * F)
        s1 = agg4[:, :F]
        s2 = agg4[:, F:2 * F]
        mean = jnp.where(has, s1 / denom, 0.0)
        var = jnp.where(has, s2 / denom - mean ** 2, 0.0)
        std = jnp.sqrt(jnp.maximum(var, 0.0) + 1e-5)
        mn = jnp.where(has, agg4[:, 2 * F:3 * F], 0.0)
        mx = jnp.where(has, agg4[:, 3 * F:], 0.0)
        agg = jnp.concatenate([mean, mn, mx, std], axis=-1)
        P0 = _mm(agg, postW[i][F:F + 4 * F], zerob, 1000)
        P1 = _mm(agg, postW[i][F + 4 * F:F + 8 * F], zerob, 1000)
        P2 = _mm(agg, postW[i][F + 8 * F:], zerob, 1000)
        out = _mm(x, postW[i][:F], postb[i], 1000) + P0 + amp * P1 + att * P2
        c = _mm(out, linW[i], linb[i], 1000)
        mu = c.mean(0)
        v = ((c - mu) ** 2).mean(0)
        cbn = (c - mu) / jnp.sqrt(v + 1e-5) * bn_g[i] + bn_b[i]
        x = (x + jax.nn.relu(cbn)) / 2.0
        S = _mm(x, e1W[i][:F], zerob, 1000)
        D = _mm(x, e1W[i][F:2 * F], zerob, 1000)
        G = _mm(ea, e1W[i][2 * F:], e1b[i], 2000)
        T = _edge_map(S, D, G, src, dst, relu=True)
        ea = ea + _mm(T, e2W[i], e2b[i], 2000) * 0.5

    return (x, pea, nea)


# speedup vs baseline: 7.4354x; 1.0016x over previous
import functools

import jax
import jax.numpy as jnp
import numpy as np
from jax import lax
from jax.experimental import pallas as pl
from jax.experimental.pallas import tpu as pltpu
from jax.experimental.pallas import tpu_sc as plsc

_N = 10000
_E = 320000
_F = 128
_AVG_LOG = float(np.log(33.0))

_NC = 2
_NS = 16
_NW = _NC * _NS
_EPT = _E // _NW
_CHM = 40


def _mm_kernel(x_ref, w_ref, b_ref, o_ref):
    o_ref[...] = jnp.dot(x_ref[...], w_ref[...],
                         preferred_element_type=jnp.float32) + b_ref[...]


def _mm(x, w, b, block_rows):
    m, k = x.shape
    n = w.shape[1]
    assert m % block_rows == 0, (m, block_rows)
    return pl.pallas_call(
        _mm_kernel,
        grid=(m // block_rows,),
        in_specs=[
            pl.BlockSpec((block_rows, k), lambda i: (i, 0)),
            pl.BlockSpec((k, n), lambda i: (0, 0)),
            pl.BlockSpec((n,), lambda i: (0,)),
        ],
        out_specs=pl.BlockSpec((block_rows, n), lambda i: (i, 0)),
        out_shape=jax.ShapeDtypeStruct((m, n), jnp.float32),
    )(x, w, b)


def _edge_map_body(relu, a_hbm, b_hbm, c_hbm, ai_hbm, bi_hbm, out_hbm,
                   ai_v0, bi_v0, a_r0, b_r0, c_r0, t_r0,
                   ai_v1, bi_v1, a_r1, b_r1, c_r1, t_r1,
                   si0, si1, sg0, sg1, sc0, sc1, so0, so1):
    wid = lax.axis_index("s") * _NC + lax.axis_index("c")
    base = wid * _EPT
    NCH = _EPT // _CHM
    sets = ((ai_v0, bi_v0, a_r0, b_r0, c_r0, t_r0, si0, sg0, sc0, so0),
            (ai_v1, bi_v1, a_r1, b_r1, c_r1, t_r1, si1, sg1, sc1, so1))

    def issue(k, st):
        ai_v, bi_v, a_r, b_r, c_r, t_r, si, sg, sc, so = st
        cb = base + k * _CHM
        i1 = pltpu.async_copy(ai_hbm.at[pl.ds(cb, _CHM)], ai_v, si)
        i2 = pltpu.async_copy(bi_hbm.at[pl.ds(cb, _CHM)], bi_v, sg)
        pltpu.async_copy(c_hbm.at[pl.ds(cb, _CHM)], c_r, sc)
        i1.wait()
        i2.wait()
        pltpu.async_copy(a_hbm.at[ai_v], a_r, si)
        pltpu.async_copy(b_hbm.at[bi_v], b_r, sg)

    def compute(k, st, first):
        ai_v, bi_v, a_r, b_r, c_r, t_r, si, sg, sc, so = st
        cb = base + k * _CHM
        pltpu.make_async_copy(a_hbm.at[ai_v], a_r, si).wait()
        pltpu.make_async_copy(b_hbm.at[bi_v], b_r, sg).wait()
        pltpu.make_async_copy(c_hbm.at[pl.ds(cb, _CHM)], c_r, sc).wait()

        @pl.when(jnp.logical_not(first))
        def _():
            pltpu.make_async_copy(
                t_r, out_hbm.at[pl.ds(cb, _CHM)], so).wait()

        def edge(j, carry2):
            for g in range(_F // 16):
                sl = pl.ds(g * 16, 16)
                t = a_r[j, sl] + b_r[j, sl] + c_r[j, sl]
                if relu:
                    t = jnp.maximum(t, 0.0)
                t_r[j, sl] = t
            return carry2

        lax.fori_loop(0, _CHM, edge, 0, unroll=2)
        pltpu.async_copy(t_r, out_hbm.at[pl.ds(cb, _CHM)], so)

    issue(0, sets[0])

    def pair(p, carry):
        for b in range(2):
            k = p * 2 + b
            nxt = k + 1

            @pl.when(nxt < NCH)
            def _():
                issue(nxt, sets[1 - b])
            compute(k, sets[b], first=(k < 2))
        return carry

    lax.fori_loop(0, NCH // 2, pair, 0)
    pltpu.make_async_copy(
        t_r0, out_hbm.at[pl.ds(base, _CHM)], so0).wait()
    pltpu.make_async_copy(
        t_r1, out_hbm.at[pl.ds(base, _CHM)], so1).wait()


def _edge_map(a, b, c, ai, bi, relu):
    mesh = plsc.VectorSubcoreMesh(core_axis_name="c", subcore_axis_name="s")
    buf = lambda: pltpu.VMEM((_CHM, _F), jnp.float32)
    idx = lambda: pltpu.VMEM((_CHM,), jnp.int32)
    return pl.kernel(
        functools.partial(_edge_map_body, relu),
        mesh=mesh,
        out_type=jax.ShapeDtypeStruct((_E, _F), jnp.float32),
        scratch_types=[
            idx(), idx(), buf(), buf(), buf(), buf(),
            idx(), idx(), buf(), buf(), buf(), buf(),
        ] + [pltpu.SemaphoreType.DMA] * 8,
    )(a, b, c, ai, bi)


_CHA = 200
_MAXCH = _E // _CHA
_BIG = 3.0e38


def _agg_body(a_hbm, b_hbm, c_hbm, src_hbm, dst_hbm, estart_hbm,
              agg_hbm, estart_v, dst_v0, src_v0, b_buf0, c_buf0,
              dst_v1, src_v1, b_buf1, c_buf1, a_buf, out_buf,
              sd0, ss0, sc0, sd1, ss1, sc1):
    wid = lax.axis_index("s") * _NC + lax.axis_index("c")
    pltpu.sync_copy(estart_hbm, estart_v)
    ev = estart_v[pl.ds(wid, 16)]
    e0 = ev[0]
    e1 = ev[1]
    NG = _F // 16
    W = 4 * _F
    sets = ((dst_v0, src_v0, b_buf0, c_buf0, sd0, ss0, sc0),
            (dst_v1, src_v1, b_buf1, c_buf1, sd1, ss1, sc1))

    def cbase(k):
        return jnp.minimum(cb0 + k * _CHA, jnp.int32(_E - _CHA))

    def issue_idx(cb, st):
        dst_v, src_v, b_buf, c_buf, sd, ss, sc = st
        cb = pl.multiple_of(cb, 8)
        pltpu.async_copy(dst_hbm.at[pl.ds(cb, _CHA)],
                         dst_v.at[pl.ds(0, _CHA)], sd)
        pltpu.async_copy(src_hbm.at[pl.ds(cb, _CHA)], src_v, ss)
        pltpu.async_copy(c_hbm.at[pl.ds(cb, _CHA)], c_buf, sc)

    def issue_gather(st):
        dst_v, src_v, b_buf, c_buf, sd, ss, sc = st
        pltpu.make_async_copy(src_hbm.at[pl.ds(0, _CHA)], src_v, ss).wait()
        pltpu.async_copy(b_hbm.at[src_v], b_buf, ss)

    def wait_chunk(cb, st):
        dst_v, src_v, b_buf, c_buf, sd, ss, sc = st
        cb = pl.multiple_of(cb, 8)
        pltpu.make_async_copy(dst_hbm.at[pl.ds(cb, _CHA)],
                              dst_v.at[pl.ds(0, _CHA)], sd).wait()
        pltpu.make_async_copy(b_hbm.at[src_v], b_buf, ss).wait()
        pltpu.make_async_copy(c_hbm.at[pl.ds(cb, _CHA)], c_buf, sc).wait()

    def init_window():
        def iw(r, c):
            z = jnp.zeros((16,), jnp.float32)
            b = jnp.full((16,), _BIG, jnp.float32)
            for g in range(NG):
                out_buf[pl.ds(r * W + g * 16, 16)] = z
                out_buf[pl.ds(r * W + _F + g * 16, 16)] = z
                out_buf[pl.ds(r * W + 2 * _F + g * 16, 16)] = b
                out_buf[pl.ds(r * W + 3 * _F + g * 16, 16)] = -b
            return c
        lax.fori_loop(0, 16, iw, 0)

    def flush_window(wb):
        pltpu.sync_copy(out_buf,
                        agg_hbm.at[pl.ds(pl.multiple_of(wb * W, 8), 16 * W)])

    def load_awin(wb):
        pltpu.sync_copy(a_hbm.at[pl.ds(pl.multiple_of(wb, 8), 16)], a_buf)

    cb0 = jnp.minimum(e0 & jnp.int32(~7), jnp.int32(_E - _CHA))
    issue_idx(cb0, sets[0])
    issue_gather(sets[0])
    wait_chunk(cb0, sets[0])
    wb0 = dst_v0[pl.ds(e0 - cb0, 16)][0] & jnp.int32(~15)
    load_awin(wb0)
    init_window()

    def do_chunk(k, b, carry):
        cb = cbase(k)
        active = (cb0 + k * _CHA) < e1

        def run(c):
            lo, wb = c
            dst_v, src_v, b_buf, c_buf, sd, ss, sc = sets[b]
            nxt_active = (cb0 + (k + 1) * _CHA) < e1

            @pl.when(nxt_active)
            def _():
                issue_idx(cbase(k + 1), sets[1 - b])

            @pl.when(k > 0)
            def _():
                wait_chunk(cb, sets[b])

            def edge(j, wb):
                eidx = cb + j
                valid = (eidx >= lo) & (eidx < e1)
                nj = jnp.where(valid, dst_v[pl.ds(j, 16)][0], wb)
                nwb = nj & jnp.int32(~15)
                adv = nwb != wb

                @pl.when(adv)
                def _():
                    flush_window(wb)
                    init_window()
                    load_awin(nwb)

                wb2 = jnp.where(adv, nwb, wb)
                r = nj - wb2
                for g in range(NG):
                    sl = pl.ds(g * 16, 16)
                    m = a_buf[r, sl] + b_buf[j, sl] + c_buf[j, sl]
                    ms = jnp.where(valid, m, 0.0)
                    mv = jnp.where(valid, m, _BIG)
                    mw = jnp.where(valid, m, -_BIG)
                    o0 = pl.ds(r * W + g * 16, 16)
                    o1 = pl.ds(r * W + _F + g * 16, 16)
                    o2 = pl.ds(r * W + 2 * _F + g * 16, 16)
                    o3 = pl.ds(r * W + 3 * _F + g * 16, 16)
                    plsc.addupdate(out_buf.at[o0], ms)
                    plsc.addupdate(out_buf.at[o1], ms * m)
                    out_buf[o2] = jnp.minimum(out_buf[o2], mv)
                    out_buf[o3] = jnp.maximum(out_buf[o3], mw)
                return wb2

            wb = lax.fori_loop(0, _CHA, edge, wb)

            @pl.when(nxt_active)
            def _():
                issue_gather(sets[1 - b])
            return (jnp.maximum(lo, cb + _CHA), wb)

        return lax.cond(active, run, lambda c: c, carry)

    def pair(p, carry):
        carry = do_chunk(p * 2, 0, carry)
        carry = do_chunk(p * 2 + 1, 1, carry)
        return carry

    lo, wb = lax.fori_loop(0, _MAXCH // 2, pair, (e0, wb0))

    @pl.when(e1 > e0)
    def _():
        flush_window(wb)


def _segment_agg(a, b, c, src_s, dst_s, estart_p):
    mesh = plsc.VectorSubcoreMesh(core_axis_name="c", subcore_axis_name="s")
    return pl.kernel(
        _agg_body,
        mesh=mesh,
        out_type=jax.ShapeDtypeStruct((_N * 4 * _F,), jnp.float32),
        scratch_types=[
            pltpu.VMEM((56,), jnp.int32),
            pltpu.VMEM((_CHA + 16,), jnp.int32),
            pltpu.VMEM((_CHA,), jnp.int32),
            pltpu.VMEM((_CHA, _F), jnp.float32),
            pltpu.VMEM((_CHA, _F), jnp.float32),
            pltpu.VMEM((_CHA + 16,), jnp.int32),
            pltpu.VMEM((_CHA,), jnp.int32),
            pltpu.VMEM((_CHA, _F), jnp.float32),
            pltpu.VMEM((_CHA, _F), jnp.float32),
            pltpu.VMEM((16, _F), jnp.float32),
            pltpu.VMEM((16 * 4 * _F,), jnp.float32),
        ] + [pltpu.SemaphoreType.DMA] * 6,
    )(a, b, c, src_s, dst_s, estart_p)


def kernel(x, edge_index, edge_attr, pos_edge_index, pos_edge_attr,
           neg_edge_index, neg_edge_attr, node_W, node_b, edge_W, edge_b,
           preW, preb, postW, postb, linW, linb, bn_g, bn_b, e1W, e1b,
           e2W, e2b):
    F = _F
    src, dst = edge_index[0], edge_index[1]
    n = x.shape[0]

    perm = jnp.argsort(dst)
    src, dst = src[perm], dst[perm]
    edge_attr = edge_attr[perm]
    offsets = jnp.searchsorted(dst, jnp.arange(n + 1, dtype=jnp.int32),
                               method='scan_unrolled').astype(jnp.int32)
    cnt = (offsets[1:] - offsets[:-1]).astype(jnp.float32)
    tgt = (jnp.arange(_NW + 1, dtype=jnp.int32) * (_E // _NW)).astype(jnp.int32)
    nstart = jnp.searchsorted(offsets, tgt, method='scan_unrolled')
    nstart = jnp.clip(((nstart + 8) // 16) * 16, 0, n).astype(jnp.int32)
    nstart = nstart.at[0].set(0).at[_NW].set(n)
    estart = offsets[nstart]
    estart_p = jnp.concatenate([estart, jnp.zeros((23,), jnp.int32)])

    x = _mm(x, node_W, node_b, 1000)
    ea = _mm(edge_attr, edge_W, edge_b, 2000)
    pea = _mm(pos_edge_attr, edge_W, edge_b, 1000)
    nea = _mm(neg_edge_attr, edge_W, edge_b, 1000)

    denom = jnp.clip(cnt, 1.0)[:, None]
    has = (cnt > 0)[:, None]
    amp = jnp.log(denom + 1.0) / _AVG_LOG
    att = _AVG_LOG / jnp.log(denom + 1.0)
    zerob = jnp.zeros((F,), jnp.float32)

    for i in range(2):
        A = _mm(x, preW[i][:F], zerob, 1000)
        B = _mm(x, preW[i][F:2 * F], zerob, 1000)
        C = _mm(ea, preW[i][2 * F:], preb[i], 2000)
        agg4 = _segment_agg(A, B, C, src, dst, estart_p).reshape(n, 4 * F)
        s1 = agg4[:, :F]
        s2 = agg4[:, F:2 * F]
        mean = jnp.where(has, s1 / denom, 0.0)
        var = jnp.where(has, s2 / denom - mean ** 2, 0.0)
        std = jnp.sqrt(jnp.maximum(var, 0.0) + 1e-5)
        mn = jnp.where(has, agg4[:, 2 * F:3 * F], 0.0)
        mx = jnp.where(has, agg4[:, 3 * F:], 0.0)
        agg = jnp.concatenate([mean, mn, mx, std], axis=-1)
        P0 = _mm(agg, postW[i][F:F + 4 * F], zerob, 1000)
        P1 = _mm(agg, postW[i][F + 4 * F:F + 8 * F], zerob, 1000)
        P2 = _mm(agg, postW[i][F + 8 * F:], zerob, 1000)
        out = _mm(x, postW[i][:F], postb[i], 1000) + P0 + amp * P1 + att * P2
        c = _mm(out, linW[i], linb[i], 1000)
        mu = c.mean(0)
        v = ((c - mu) ** 2).mean(0)
        cbn = (c - mu) / jnp.sqrt(v + 1e-5) * bn_g[i] + bn_b[i]
        x = (x + jax.nn.relu(cbn)) / 2.0
        S = _mm(x, e1W[i][:F], zerob, 1000)
        D = _mm(x, e1W[i][F:2 * F], zerob, 1000)
        G = _mm(ea, e1W[i][2 * F:], e1b[i], 2000)
        T = _edge_map(S, D, G, src, dst, relu=True)
        ea = ea + _mm(T, e2W[i], e2b[i], 2000) * 0.5

    return (x, pea, nea)
